# baseline (speedup 1.0000x reference)
import jax
import jax.numpy as jnp
from jax import lax
from jax.experimental import pallas as pl
from jax.experimental.pallas import tpu as pltpu

_B = 64
_CIN = 512
_CMID = 1024
_G = 19
_P = 21
_XCOLS = 504
_NPOS = 448
_K9 = 9 * _CIN
_NATTR = 85
_DWROWS = 384
_LEAKY = 0.1
_IPG = 8


def _body(xf_ref, wf_ref, cb_ref, dw_ref, db_ref, anc_ref, out_ref,
          h_ref, xb_ref):
    xb_ref[...] = xf_ref[...].astype(jnp.bfloat16)
    im2col = jnp.concatenate(
        [jnp.concatenate(
            [xb_ref[i, :,
                    (t // 3) * _P + (t % 3):(t // 3) * _P + (t % 3) + _NPOS]
             for t in range(9)], axis=0)
         for i in range(_IPG)], axis=1)
    acc = jnp.dot(wf_ref[...], im2col,
                  preferred_element_type=jnp.float32) + cb_ref[...]
    h_ref[...] = jnp.maximum(acc, _LEAKY * acc).astype(jnp.bfloat16)

    o2 = jnp.dot(dw_ref[...], h_ref[...],
                 preferred_element_type=jnp.float32) + db_ref[...]

    nw = _IPG * _NPOS
    jidx = lax.broadcasted_iota(jnp.int32, (1, nw), 1) % _NPOS
    gxf = (jidx % _P).astype(jnp.float32)
    gyf = (jidx // _P).astype(jnp.float32)
    gs = jnp.float32(_G)
    zeros2 = jnp.zeros((2, nw), jnp.float32)

    for a in range(3):
        r = 128 * a
        logits = o2[r:r + 80, :]
        obj = jax.nn.sigmoid(o2[r + 80:r + 81, :])
        lmax = jnp.max(logits, axis=0, keepdims=True)
        m = obj * jax.nn.sigmoid(lmax)
        io = lax.broadcasted_iota(jnp.int32, (80, nw), 0)
        lab = jnp.min(jnp.where(logits >= lmax, io, 10000),
                      axis=0, keepdims=True).astype(jnp.float32)
        tx = jax.nn.sigmoid(o2[r + 81:r + 82, :])
        ty = jax.nn.sigmoid(o2[r + 82:r + 83, :])
        tw = jax.nn.sigmoid(o2[r + 83:r + 84, :])
        th = jax.nn.sigmoid(o2[r + 84:r + 85, :])
        xc = (tx + gxf) / gs
        yc = (ty + gyf) / gs
        wd = jnp.exp(tw) * anc_ref[a, 0]
        ht = jnp.exp(th) * anc_ref[a, 1]
        full = jnp.concatenate([m, xc, yc, wd, ht, lab, zeros2], axis=0)
        for i in range(_IPG):
            out_ref[i, a] = full[:, i * _NPOS:(i + 1) * _NPOS]


def kernel(x, conv_w, conv_b, detect_w, detect_b, anchors):
    xf = jnp.pad(x, ((0, 0), (0, 0), (1, 4), (1, 1))).reshape(_B, _CIN, _XCOLS)
    wf = conv_w.transpose(0, 2, 3, 1).reshape(_CMID, _K9).astype(jnp.bfloat16)
    cb = conv_b.reshape(_CMID, 1)
    dwm = detect_w[:, :, 0, 0].reshape(3, _NATTR, _CMID)
    dbm = detect_b.reshape(3, _NATTR, 1)
    pad43 = jnp.zeros((3, 43, _CMID), jnp.float32)
    pad43b = jnp.zeros((3, 43, 1), jnp.float32)
    dwp = jnp.concatenate([dwm[:, 5:], dwm[:, :5], pad43], axis=1)
    dbp = jnp.concatenate([dbm[:, 5:], dbm[:, :5], pad43b], axis=1)
    dwp = dwp.reshape(_DWROWS, _CMID).astype(jnp.bfloat16)
    dbp = dbp.reshape(_DWROWS, 1)

    def _run(xfs, wfs, cbs, dwps, dbps, ancs):
        nb = xfs.shape[0]
        return pl.pallas_call(
            _body,
            grid=(nb // _IPG,),
            in_specs=[
                pl.BlockSpec((_IPG, _CIN, _XCOLS), lambda b: (b, 0, 0)),
                pl.BlockSpec((_CMID, _K9), lambda b: (0, 0)),
                pl.BlockSpec((_CMID, 1), lambda b: (0, 0)),
                pl.BlockSpec((_DWROWS, _CMID), lambda b: (0, 0)),
                pl.BlockSpec((_DWROWS, 1), lambda b: (0, 0)),
                pl.BlockSpec(memory_space=pltpu.SMEM),
            ],
            out_specs=pl.BlockSpec((_IPG, 3, 8, _NPOS),
                                   lambda b: (b, 0, 0, 0)),
            out_shape=jax.ShapeDtypeStruct((nb, 3, 8, _NPOS), jnp.float32),
            scratch_shapes=[
                pltpu.VMEM((_CMID, _IPG * _NPOS), jnp.bfloat16),
                pltpu.VMEM((_IPG, _CIN, _XCOLS), jnp.bfloat16),
            ],
            compiler_params=pltpu.CompilerParams(
                dimension_semantics=("parallel",),
                vmem_limit_bytes=60000 * 1024,
            ),
            name="detection_layer",
        )(xfs, wfs, cbs, dwps, dbps, ancs)

    out_full = _run(xf, wf, cb, dwp, dbp, anchors)

    o = out_full[:, :, :6, :441].reshape(_B, 3, 6, _P, _P)[:, :, :, :_G, :_G]
    return o.transpose(0, 3, 4, 1, 2).reshape(_B, _G * _G * 3, 6)


# speedup vs baseline: 1.0321x; 1.0321x over previous
import jax
import jax.numpy as jnp
from jax import lax
from jax.experimental import pallas as pl
from jax.experimental.pallas import tpu as pltpu

_B = 64
_CIN = 512
_CMID = 1024
_G = 19
_P = 21
_XCOLS = 504
_NPOS = 448
_K9 = 9 * _CIN
_NATTR = 85
_DWROWS = 384
_LEAKY = 0.1
_IPG = 4


def _body(xf_ref, wf_ref, cb_ref, dw_ref, db_ref, anc_ref, out_ref,
          h_ref, xb_ref):
    xb_ref[...] = xf_ref[...].astype(jnp.bfloat16)
    im2col = jnp.concatenate(
        [jnp.concatenate(
            [xb_ref[i, :,
                    (t // 3) * _P + (t % 3):(t // 3) * _P + (t % 3) + _NPOS]
             for t in range(9)], axis=0)
         for i in range(_IPG)], axis=1)
    acc = jnp.dot(wf_ref[...], im2col,
                  preferred_element_type=jnp.float32) + cb_ref[...]
    h_ref[...] = jnp.maximum(acc, _LEAKY * acc).astype(jnp.bfloat16)

    o2 = jnp.dot(dw_ref[...], h_ref[...],
                 preferred_element_type=jnp.float32) + db_ref[...]

    nw = _IPG * _NPOS
    jidx = lax.broadcasted_iota(jnp.int32, (1, nw), 1) % _NPOS
    gxf = (jidx % _P).astype(jnp.float32)
    gyf = (jidx // _P).astype(jnp.float32)
    gs = jnp.float32(_G)
    zeros2 = jnp.zeros((2, nw), jnp.float32)

    for a in range(3):
        r = 128 * a
        logits = o2[r:r + 80, :]
        obj = jax.nn.sigmoid(o2[r + 80:r + 81, :])
        lmax = jnp.max(logits, axis=0, keepdims=True)
        m = obj * jax.nn.sigmoid(lmax)
        io = lax.broadcasted_iota(jnp.int32, (80, nw), 0)
        lab = jnp.min(jnp.where(logits >= lmax, io, 10000),
                      axis=0, keepdims=True).astype(jnp.float32)
        tx = jax.nn.sigmoid(o2[r + 81:r + 82, :])
        ty = jax.nn.sigmoid(o2[r + 82:r + 83, :])
        tw = jax.nn.sigmoid(o2[r + 83:r + 84, :])
        th = jax.nn.sigmoid(o2[r + 84:r + 85, :])
        xc = (tx + gxf) / gs
        yc = (ty + gyf) / gs
        wd = jnp.exp(tw) * anc_ref[a, 0]
        ht = jnp.exp(th) * anc_ref[a, 1]
        full = jnp.concatenate([m, xc, yc, wd, ht, lab, zeros2], axis=0)
        for i in range(_IPG):
            out_ref[i, a] = full[:, i * _NPOS:(i + 1) * _NPOS]


def kernel(x, conv_w, conv_b, detect_w, detect_b, anchors):
    xf = jnp.pad(x, ((0, 0), (0, 0), (1, 4), (1, 1))).reshape(_B, _CIN, _XCOLS)
    wf = conv_w.transpose(0, 2, 3, 1).reshape(_CMID, _K9).astype(jnp.bfloat16)
    cb = conv_b.reshape(_CMID, 1)
    dwm = detect_w[:, :, 0, 0].reshape(3, _NATTR, _CMID)
    dbm = detect_b.reshape(3, _NATTR, 1)
    pad43 = jnp.zeros((3, 43, _CMID), jnp.float32)
    pad43b = jnp.zeros((3, 43, 1), jnp.float32)
    dwp = jnp.concatenate([dwm[:, 5:], dwm[:, :5], pad43], axis=1)
    dbp = jnp.concatenate([dbm[:, 5:], dbm[:, :5], pad43b], axis=1)
    dwp = dwp.reshape(_DWROWS, _CMID).astype(jnp.bfloat16)
    dbp = dbp.reshape(_DWROWS, 1)

    def _run(xfs, wfs, cbs, dwps, dbps, ancs):
        nb = xfs.shape[0]
        return pl.pallas_call(
            _body,
            grid=(nb // _IPG,),
            in_specs=[
                pl.BlockSpec((_IPG, _CIN, _XCOLS), lambda b: (b, 0, 0)),
                pl.BlockSpec((_CMID, _K9), lambda b: (0, 0)),
                pl.BlockSpec((_CMID, 1), lambda b: (0, 0)),
                pl.BlockSpec((_DWROWS, _CMID), lambda b: (0, 0)),
                pl.BlockSpec((_DWROWS, 1), lambda b: (0, 0)),
                pl.BlockSpec(memory_space=pltpu.SMEM),
            ],
            out_specs=pl.BlockSpec((_IPG, 3, 8, _NPOS),
                                   lambda b: (b, 0, 0, 0)),
            out_shape=jax.ShapeDtypeStruct((nb, 3, 8, _NPOS), jnp.float32),
            scratch_shapes=[
                pltpu.VMEM((_CMID, _IPG * _NPOS), jnp.bfloat16),
                pltpu.VMEM((_IPG, _CIN, _XCOLS), jnp.bfloat16),
            ],
            compiler_params=pltpu.CompilerParams(
                dimension_semantics=("parallel",),
                vmem_limit_bytes=60000 * 1024,
            ),
            name="detection_layer",
        )(xfs, wfs, cbs, dwps, dbps, ancs)

    out_full = _run(xf, wf, cb, dwp, dbp, anchors)

    o = out_full[:, :, :6, :441].reshape(_B, 3, 6, _P, _P)[:, :, :, :_G, :_G]
    return o.transpose(0, 3, 4, 1, 2).reshape(_B, _G * _G * 3, 6)
